# baseline (speedup 1.0000x reference)
import functools

import jax
import jax.numpy as jnp
from jax import lax
from jax.experimental import pallas as pl
from jax.experimental.pallas import tpu as pltpu
from jax.experimental.pallas import tpu_sc as plsc

BINSIZE = 500
N_BINS = 200
NC = 2
NS = 16
NW = NC * NS
L = 16

CHUNK = 2048
TAIL = 576
N_FRAG = 488 * CHUNK + TAIL

DIV_M = 33555
DIV_S = 22

_SC_PARAMS = pltpu.CompilerParams(
    use_tc_tiling_on_sc=False, needs_layout_passes=False)


def _sc_gather_rows(table, idx):
    b = idx.shape[0]
    d = table.shape[1]
    b_per_w = b // NW
    mesh = plsc.VectorSubcoreMesh(
        core_axis_name="c", subcore_axis_name="s", num_cores=NC, num_subcores=NS
    )

    @functools.partial(
        pl.kernel,
        mesh=mesh,
        out_type=jax.ShapeDtypeStruct((b, d), jnp.float32),
        compiler_params=_SC_PARAMS,
        scratch_types=[
            pltpu.VMEM((b_per_w,), jnp.int32),
            pltpu.VMEM((b_per_w, d), jnp.float32),
            pltpu.SemaphoreType.DMA,
        ],
    )
    def k(table_hbm, idx_hbm, out_hbm, idx_v, rows_v, sem):
        wid = lax.axis_index("s") * NC + lax.axis_index("c")
        base = wid * b_per_w
        pltpu.sync_copy(idx_hbm.at[pl.ds(base, b_per_w)], idx_v)
        pltpu.async_copy(table_hbm.at[idx_v], rows_v, sem).wait()
        pltpu.sync_copy(rows_v, out_hbm.at[pl.ds(base, b_per_w)])

    return k(table, idx)


def _sc_extract_columns(table_t, rsorted, tail_cols):
    d = table_t.shape[0]
    nreg = table_t.shape[1]
    ntile = (nreg // 128) * 128
    ntail = nreg - ntile
    b = rsorted.shape[0]
    b_per_w = b // NW
    mesh = plsc.VectorSubcoreMesh(
        core_axis_name="c", subcore_axis_name="s", num_cores=NC, num_subcores=NS
    )

    @functools.partial(
        pl.kernel,
        mesh=mesh,
        out_type=jax.ShapeDtypeStruct((b, 256), jnp.float32),
        compiler_params=pltpu.CompilerParams(
            use_tc_tiling_on_sc=True, needs_layout_passes=False),
        scratch_types=[
            pltpu.VMEM((b_per_w,), jnp.int32),
            pltpu.VMEM((d, 128), jnp.float32),
            pltpu.VMEM((ntail, d), jnp.float32),
            pltpu.VMEM((b_per_w, 128), jnp.float32),
            pltpu.VMEM((b_per_w, 128), jnp.float32),
            pltpu.SemaphoreType.DMA,
        ],
    )
    def k(tab_hbm, idx_hbm, tail_hbm, out_hbm, idx_v, band, tail_v,
          xa, xb, sem):
        wid = lax.axis_index("s") * NC + lax.axis_index("c")
        base = wid * b_per_w
        pltpu.sync_copy(idx_hbm.at[pl.ds(base, b_per_w)], idx_v)
        pltpu.sync_copy(tail_hbm, tail_v)
        iota = lax.broadcasted_iota(jnp.int32, (L,), 0)
        zeros = iota * 0

        def body(s, band_prev):
            r = jnp.max(plsc.load_gather(idx_v, [zeros + s]))
            bd = r >> 7

            @pl.when(jnp.logical_and(bd != band_prev, r < ntile))
            def _():
                pltpu.sync_copy(
                    tab_hbm.at[pl.ds(0, d), pl.ds(bd * 128, 128)], band)

            @pl.when(r < ntile)
            def _():
                cvec = zeros + (r & 127)
                for kk in range(13):
                    row0 = min(kk * L, d - L)
                    v = plsc.load_gather(band, [row0 + iota, cvec])
                    if row0 < 128:
                        xa[s, pl.ds(row0, L)] = v
                    else:
                        xb[s, pl.ds(row0 - 128, L)] = v

            @pl.when(r >= ntile)
            def _():
                tr = r - ntile
                for kk in range(13):
                    row0 = min(kk * L, d - L)
                    v = plsc.load_gather(tail_v, [zeros + tr, row0 + iota])
                    if row0 < 128:
                        xa[s, pl.ds(row0, L)] = v
                    else:
                        xb[s, pl.ds(row0 - 128, L)] = v

            return jnp.where(r < ntile, bd, band_prev)

        lax.fori_loop(0, b_per_w, body, jnp.int32(-1))
        pltpu.sync_copy(xa, out_hbm.at[pl.ds(base, b_per_w), pl.ds(0, 128)])
        pltpu.sync_copy(xb, out_hbm.at[pl.ds(base, b_per_w), pl.ds(128, 128)])

    return k(table_t, rsorted, tail_cols)


def _tc_log_softmax(rows, d):
    n, dp = rows.shape

    def body(x_ref, o_ref):
        x = x_ref[:, :d]
        m = jnp.max(x, axis=1, keepdims=True)
        e = jnp.exp(x - m)
        s = jnp.sum(e, axis=1, keepdims=True)
        o_ref[...] = (x - m) - jnp.log(s) + jnp.log(jnp.float32(d))

    return pl.pallas_call(
        body,
        out_shape=jax.ShapeDtypeStruct((n, d), jnp.float32),
    )(rows)


N_FPAD = 1048576


def _tc_flat_indices(c0, lrix, window):
    n = c0.shape[0]

    def body(w_ref, c_ref, l_ref, o_ref):
        w = w_ref[0]
        x = c_ref[...] - w
        bins = ((x >> 2) * DIV_M) >> DIV_S
        o_ref[pl.ds(0, n)] = l_ref[...] * N_BINS + bins
        o_ref[pl.ds(n, N_FPAD - n)] = jnp.zeros((N_FPAD - n,), jnp.int32)

    return pl.pallas_call(
        body,
        in_specs=[
            pl.BlockSpec(memory_space=pltpu.SMEM),
            pl.BlockSpec((n,), lambda: (0,)),
            pl.BlockSpec((n,), lambda: (0,)),
        ],
        out_specs=pl.BlockSpec((N_FPAD,), lambda: (0,)),
        out_shape=jax.ShapeDtypeStruct((N_FPAD,), jnp.int32),
    )(window, c0, lrix)


def _sc_fragment_gather(heights_flat, findices):
    mesh = plsc.VectorSubcoreMesh(
        core_axis_name="c", subcore_axis_name="s", num_cores=NC, num_subcores=NS
    )

    nchunk = N_FPAD // (NW * CHUNK)

    @functools.partial(
        pl.kernel,
        mesh=mesh,
        out_type=jax.ShapeDtypeStruct((N_FPAD,), jnp.float32),
        compiler_params=_SC_PARAMS,
        scratch_types=[
            pltpu.VMEM((2, CHUNK), jnp.int32),
            pltpu.VMEM((2, CHUNK), jnp.float32),
            pltpu.SemaphoreType.DMA,
            pltpu.SemaphoreType.DMA,
            pltpu.SemaphoreType.DMA,
        ],
    )
    def k(h_hbm, f_hbm, out_hbm, fbuf, obuf, sem_i, sem_g, sem_o):
        wid = lax.axis_index("s") * NC + lax.axis_index("c")
        base = wid * (nchunk * CHUNK)

        def in_copy(j, p):
            return pltpu.make_async_copy(
                f_hbm.at[pl.ds(base + j * CHUNK, CHUNK)], fbuf.at[p], sem_i)

        def out_copy(j, p):
            return pltpu.make_async_copy(
                obuf.at[p], out_hbm.at[pl.ds(base + j * CHUNK, CHUNK)], sem_o)

        in_copy(0, 0).start()
        for j in range(nchunk):
            p = j & 1
            in_copy(j, p).wait()
            if j + 1 < nchunk:
                in_copy(j + 1, 1 - p).start()
            if j >= 2:
                out_copy(j - 2, p).wait()
            pltpu.async_copy(h_hbm.at[fbuf.at[p]], obuf.at[p], sem_g).wait()
            out_copy(j, p).start()
        out_copy(nchunk - 2, (nchunk - 2) & 1).wait()
        out_copy(nchunk - 1, (nchunk - 1) & 1).wait()

    return k(heights_flat, findices)


def kernel(regions_oi, coordinates, local_region_ix, window, baseline_weight):
    table_t = baseline_weight.T
    ntile = (table_t.shape[1] // 128) * 128
    tail_cols = baseline_weight[ntile:, :]
    order = jnp.argsort(regions_oi)
    rsorted = jnp.take(regions_oi, order)
    pos = jnp.zeros_like(order).at[order].set(
        jnp.arange(order.shape[0], dtype=order.dtype))
    xsel = _sc_extract_columns(table_t, rsorted, tail_cols)
    heights_sorted = _tc_log_softmax(xsel, N_BINS)
    heights = _sc_gather_rows(heights_sorted, pos)
    heights_flat = heights.reshape(-1)
    c0 = coordinates[:, 0]
    findices = _tc_flat_indices(c0, local_region_ix, window)
    out = _sc_fragment_gather(heights_flat, findices)
    return out[:coordinates.shape[0]]


# speedup vs baseline: 1.9578x; 1.9578x over previous
import functools

import jax
import jax.numpy as jnp
from jax import lax
from jax.experimental import pallas as pl
from jax.experimental.pallas import tpu as pltpu
from jax.experimental.pallas import tpu_sc as plsc

BINSIZE = 500
N_BINS = 200
NC = 2
NS = 16
NW = NC * NS
L = 16

CHUNK = 4096
TAIL = 576
N_FRAG = 244 * CHUNK + TAIL

DIV_M = 33555
DIV_S = 22

_SC_PARAMS = pltpu.CompilerParams(
    use_tc_tiling_on_sc=False, needs_layout_passes=False)


def _sc_gather_rows(table, idx):
    b = idx.shape[0]
    d = table.shape[1]
    b_per_w = b // NW
    mesh = plsc.VectorSubcoreMesh(
        core_axis_name="c", subcore_axis_name="s", num_cores=NC, num_subcores=NS
    )

    @functools.partial(
        pl.kernel,
        mesh=mesh,
        out_type=jax.ShapeDtypeStruct((b, d), jnp.float32),
        compiler_params=_SC_PARAMS,
        scratch_types=[
            pltpu.VMEM((b_per_w,), jnp.int32),
            pltpu.VMEM((b_per_w, d), jnp.float32),
            pltpu.SemaphoreType.DMA,
        ],
    )
    def k(table_hbm, idx_hbm, out_hbm, idx_v, rows_v, sem):
        wid = lax.axis_index("s") * NC + lax.axis_index("c")
        base = wid * b_per_w
        pltpu.sync_copy(idx_hbm.at[pl.ds(base, b_per_w)], idx_v)
        pltpu.async_copy(table_hbm.at[idx_v], rows_v, sem).wait()
        pltpu.sync_copy(rows_v, out_hbm.at[pl.ds(base, b_per_w)])

    return k(table, idx)


def _sc_extract_columns(table_t, rsorted, tail_cols):
    d = table_t.shape[0]
    nreg = table_t.shape[1]
    ntile = (nreg // 128) * 128
    ntail = nreg - ntile
    b = rsorted.shape[0]
    b_per_w = b // NW
    mesh = plsc.VectorSubcoreMesh(
        core_axis_name="c", subcore_axis_name="s", num_cores=NC, num_subcores=NS
    )

    @functools.partial(
        pl.kernel,
        mesh=mesh,
        out_type=jax.ShapeDtypeStruct((b, 256), jnp.float32),
        compiler_params=pltpu.CompilerParams(
            use_tc_tiling_on_sc=True, needs_layout_passes=False),
        scratch_types=[
            pltpu.VMEM((b_per_w,), jnp.int32),
            pltpu.VMEM((d, 128), jnp.float32),
            pltpu.VMEM((ntail, d), jnp.float32),
            pltpu.VMEM((b_per_w, 128), jnp.float32),
            pltpu.VMEM((b_per_w, 128), jnp.float32),
            pltpu.SemaphoreType.DMA,
        ],
    )
    def k(tab_hbm, idx_hbm, tail_hbm, out_hbm, idx_v, band, tail_v,
          xa, xb, sem):
        wid = lax.axis_index("s") * NC + lax.axis_index("c")
        base = wid * b_per_w
        pltpu.sync_copy(idx_hbm.at[pl.ds(base, b_per_w)], idx_v)
        pltpu.sync_copy(tail_hbm, tail_v)
        iota = lax.broadcasted_iota(jnp.int32, (L,), 0)
        zeros = iota * 0

        def body(s, band_prev):
            r = jnp.max(plsc.load_gather(idx_v, [zeros + s]))
            bd = r >> 7

            @pl.when(jnp.logical_and(bd != band_prev, r < ntile))
            def _():
                pltpu.sync_copy(
                    tab_hbm.at[pl.ds(0, d), pl.ds(bd * 128, 128)], band)

            @pl.when(r < ntile)
            def _():
                cvec = zeros + (r & 127)
                for kk in range(13):
                    row0 = min(kk * L, d - L)
                    v = plsc.load_gather(band, [row0 + iota, cvec])
                    if row0 < 128:
                        xa[s, pl.ds(row0, L)] = v
                    else:
                        xb[s, pl.ds(row0 - 128, L)] = v

            @pl.when(r >= ntile)
            def _():
                tr = r - ntile
                for kk in range(13):
                    row0 = min(kk * L, d - L)
                    v = plsc.load_gather(tail_v, [zeros + tr, row0 + iota])
                    if row0 < 128:
                        xa[s, pl.ds(row0, L)] = v
                    else:
                        xb[s, pl.ds(row0 - 128, L)] = v

            return jnp.where(r < ntile, bd, band_prev)

        lax.fori_loop(0, b_per_w, body, jnp.int32(-1))
        pltpu.sync_copy(xa, out_hbm.at[pl.ds(base, b_per_w), pl.ds(0, 128)])
        pltpu.sync_copy(xb, out_hbm.at[pl.ds(base, b_per_w), pl.ds(128, 128)])

    return k(table_t, rsorted, tail_cols)


def _tc_log_softmax(rows, d):
    n, dp = rows.shape

    def body(x_ref, o_ref):
        x = x_ref[:, :d]
        m = jnp.max(x, axis=1, keepdims=True)
        e = jnp.exp(x - m)
        s = jnp.sum(e, axis=1, keepdims=True)
        o_ref[...] = (x - m) - jnp.log(s) + jnp.log(jnp.float32(d))

    return pl.pallas_call(
        body,
        out_shape=jax.ShapeDtypeStruct((n, d), jnp.float32),
    )(rows)


N_FPAD = 1000448


def _tc_flat_indices(c0, lrix, window):
    n = c0.shape[0]

    def body(w_ref, c_ref, l_ref, o_ref):
        w = w_ref[0]
        x = c_ref[...] - w
        bins = ((x >> 2) * DIV_M) >> DIV_S
        o_ref[pl.ds(0, n)] = l_ref[...] * N_BINS + bins

    return pl.pallas_call(
        body,
        in_specs=[
            pl.BlockSpec(memory_space=pltpu.SMEM),
            pl.BlockSpec((n,), lambda: (0,)),
            pl.BlockSpec((n,), lambda: (0,)),
        ],
        out_specs=pl.BlockSpec((N_FPAD,), lambda: (0,)),
        out_shape=jax.ShapeDtypeStruct((N_FPAD,), jnp.int32),
    )(window, c0, lrix)


def _sc_fragment_gather(heights_flat, findices):
    mesh = plsc.VectorSubcoreMesh(
        core_axis_name="c", subcore_axis_name="s", num_cores=NC, num_subcores=NS
    )

    @functools.partial(
        pl.kernel,
        mesh=mesh,
        out_type=jax.ShapeDtypeStruct((N_FPAD,), jnp.float32),
        compiler_params=_SC_PARAMS,
        scratch_types=[
            pltpu.VMEM((CHUNK,), jnp.int32),
            pltpu.VMEM((CHUNK,), jnp.float32),
            pltpu.SemaphoreType.DMA,
        ],
    )
    def k(h_hbm, f_hbm, out_hbm, fbuf, obuf, sem):
        wid = lax.axis_index("s") * NC + lax.axis_index("c")

        def do_chunk(off, n):
            pltpu.sync_copy(f_hbm.at[pl.ds(off, n)], fbuf.at[pl.ds(0, n)])
            pltpu.async_copy(h_hbm.at[fbuf.at[pl.ds(0, n)]],
                             obuf.at[pl.ds(0, n)], sem).wait()
            pltpu.sync_copy(obuf.at[pl.ds(0, n)],
                            out_hbm.at[pl.ds(off, n)])

        cbase = jnp.where(wid < 20, 8 * wid, 160 + 7 * (wid - 20))

        def main_chunk(j, _):
            do_chunk((cbase + j) * CHUNK, CHUNK)
            return 0

        lax.fori_loop(0, 7, main_chunk, 0)

        @pl.when(wid < 20)
        def _():
            do_chunk((cbase + 7) * CHUNK, CHUNK)

        @pl.when(wid == NW - 1)
        def _():
            do_chunk(244 * CHUNK, TAIL)

    return k(heights_flat, findices)


def kernel(regions_oi, coordinates, local_region_ix, window, baseline_weight):
    table_t = baseline_weight.T
    ntile = (table_t.shape[1] // 128) * 128
    tail_cols = baseline_weight[ntile:, :]
    order = jnp.argsort(regions_oi)
    rsorted = jnp.take(regions_oi, order)
    pos = jnp.zeros_like(order).at[order].set(
        jnp.arange(order.shape[0], dtype=order.dtype))
    xsel = _sc_extract_columns(table_t, rsorted, tail_cols)
    heights_sorted = _tc_log_softmax(xsel, N_BINS)
    heights = _sc_gather_rows(heights_sorted, pos)
    heights_flat = heights.reshape(-1)
    c0 = coordinates[:, 0]
    findices = _tc_flat_indices(c0, local_region_ix, window)
    out = _sc_fragment_gather(heights_flat, findices)
    return out[:coordinates.shape[0]]


# speedup vs baseline: 2.0139x; 1.0287x over previous
import functools

import jax
import jax.numpy as jnp
from jax import lax
from jax.experimental import pallas as pl
from jax.experimental.pallas import tpu as pltpu
from jax.experimental.pallas import tpu_sc as plsc

BINSIZE = 500
N_BINS = 200
NC = 2
NS = 16
NW = NC * NS
L = 16

CHUNK = 8192
TAIL = 576
N_FRAG = 122 * CHUNK + TAIL

DIV_M = 33555
DIV_S = 22

_SC_PARAMS = pltpu.CompilerParams(
    use_tc_tiling_on_sc=False, needs_layout_passes=False)


def _sc_gather_rows(table, idx):
    b = idx.shape[0]
    d = table.shape[1]
    b_per_w = b // NW
    mesh = plsc.VectorSubcoreMesh(
        core_axis_name="c", subcore_axis_name="s", num_cores=NC, num_subcores=NS
    )

    @functools.partial(
        pl.kernel,
        mesh=mesh,
        out_type=jax.ShapeDtypeStruct((b, d), jnp.float32),
        compiler_params=_SC_PARAMS,
        scratch_types=[
            pltpu.VMEM((b_per_w,), jnp.int32),
            pltpu.VMEM((b_per_w, d), jnp.float32),
            pltpu.SemaphoreType.DMA,
        ],
    )
    def k(table_hbm, idx_hbm, out_hbm, idx_v, rows_v, sem):
        wid = lax.axis_index("s") * NC + lax.axis_index("c")
        base = wid * b_per_w
        pltpu.sync_copy(idx_hbm.at[pl.ds(base, b_per_w)], idx_v)
        pltpu.async_copy(table_hbm.at[idx_v], rows_v, sem).wait()
        pltpu.sync_copy(rows_v, out_hbm.at[pl.ds(base, b_per_w)])

    return k(table, idx)


def _sc_extract_columns(table_t, rsorted, tail_cols):
    d = table_t.shape[0]
    nreg = table_t.shape[1]
    ntile = (nreg // 128) * 128
    ntail = nreg - ntile
    b = rsorted.shape[0]
    b_per_w = b // NW
    mesh = plsc.VectorSubcoreMesh(
        core_axis_name="c", subcore_axis_name="s", num_cores=NC, num_subcores=NS
    )

    @functools.partial(
        pl.kernel,
        mesh=mesh,
        out_type=jax.ShapeDtypeStruct((b, 256), jnp.float32),
        compiler_params=pltpu.CompilerParams(
            use_tc_tiling_on_sc=True, needs_layout_passes=False),
        scratch_types=[
            pltpu.VMEM((b_per_w,), jnp.int32),
            pltpu.VMEM((d, 128), jnp.float32),
            pltpu.VMEM((ntail, d), jnp.float32),
            pltpu.VMEM((b_per_w, 128), jnp.float32),
            pltpu.VMEM((b_per_w, 128), jnp.float32),
            pltpu.SemaphoreType.DMA,
        ],
    )
    def k(tab_hbm, idx_hbm, tail_hbm, out_hbm, idx_v, band, tail_v,
          xa, xb, sem):
        wid = lax.axis_index("s") * NC + lax.axis_index("c")
        base = wid * b_per_w
        pltpu.sync_copy(idx_hbm.at[pl.ds(base, b_per_w)], idx_v)
        pltpu.sync_copy(tail_hbm, tail_v)
        iota = lax.broadcasted_iota(jnp.int32, (L,), 0)
        zeros = iota * 0

        def body(s, band_prev):
            r = jnp.max(plsc.load_gather(idx_v, [zeros + s]))
            bd = r >> 7

            @pl.when(jnp.logical_and(bd != band_prev, r < ntile))
            def _():
                pltpu.sync_copy(
                    tab_hbm.at[pl.ds(0, d), pl.ds(bd * 128, 128)], band)

            @pl.when(r < ntile)
            def _():
                cvec = zeros + (r & 127)
                for kk in range(13):
                    row0 = min(kk * L, d - L)
                    v = plsc.load_gather(band, [row0 + iota, cvec])
                    if row0 < 128:
                        xa[s, pl.ds(row0, L)] = v
                    else:
                        xb[s, pl.ds(row0 - 128, L)] = v

            @pl.when(r >= ntile)
            def _():
                tr = r - ntile
                for kk in range(13):
                    row0 = min(kk * L, d - L)
                    v = plsc.load_gather(tail_v, [zeros + tr, row0 + iota])
                    if row0 < 128:
                        xa[s, pl.ds(row0, L)] = v
                    else:
                        xb[s, pl.ds(row0 - 128, L)] = v

            return jnp.where(r < ntile, bd, band_prev)

        lax.fori_loop(0, b_per_w, body, jnp.int32(-1))
        pltpu.sync_copy(xa, out_hbm.at[pl.ds(base, b_per_w), pl.ds(0, 128)])
        pltpu.sync_copy(xb, out_hbm.at[pl.ds(base, b_per_w), pl.ds(128, 128)])

    return k(table_t, rsorted, tail_cols)


def _tc_log_softmax(rows, d):
    n, dp = rows.shape

    def body(x_ref, o_ref):
        x = x_ref[:, :d]
        m = jnp.max(x, axis=1, keepdims=True)
        e = jnp.exp(x - m)
        s = jnp.sum(e, axis=1, keepdims=True)
        o_ref[...] = (x - m) - jnp.log(s) + jnp.log(jnp.float32(d))

    return pl.pallas_call(
        body,
        out_shape=jax.ShapeDtypeStruct((n, d), jnp.float32),
    )(rows)


N_FPAD = 1000448


def _tc_flat_indices(c0, lrix, window):
    n = c0.shape[0]

    def body(w_ref, c_ref, l_ref, o_ref):
        w = w_ref[0]
        x = c_ref[...] - w
        bins = ((x >> 2) * DIV_M) >> DIV_S
        o_ref[pl.ds(0, n)] = l_ref[...] * N_BINS + bins

    return pl.pallas_call(
        body,
        in_specs=[
            pl.BlockSpec(memory_space=pltpu.SMEM),
            pl.BlockSpec((n,), lambda: (0,)),
            pl.BlockSpec((n,), lambda: (0,)),
        ],
        out_specs=pl.BlockSpec((N_FPAD,), lambda: (0,)),
        out_shape=jax.ShapeDtypeStruct((N_FPAD,), jnp.int32),
    )(window, c0, lrix)


def _sc_fragment_gather(heights_flat, findices):
    mesh = plsc.VectorSubcoreMesh(
        core_axis_name="c", subcore_axis_name="s", num_cores=NC, num_subcores=NS
    )

    @functools.partial(
        pl.kernel,
        mesh=mesh,
        out_type=jax.ShapeDtypeStruct((N_FPAD,), jnp.float32),
        compiler_params=_SC_PARAMS,
        scratch_types=[
            pltpu.VMEM((CHUNK,), jnp.int32),
            pltpu.VMEM((CHUNK,), jnp.float32),
            pltpu.SemaphoreType.DMA,
        ],
    )
    def k(h_hbm, f_hbm, out_hbm, fbuf, obuf, sem):
        wid = lax.axis_index("s") * NC + lax.axis_index("c")

        def do_chunk(off, n):
            pltpu.sync_copy(f_hbm.at[pl.ds(off, n)], fbuf.at[pl.ds(0, n)])
            pltpu.async_copy(h_hbm.at[fbuf.at[pl.ds(0, n)]],
                             obuf.at[pl.ds(0, n)], sem).wait()
            pltpu.sync_copy(obuf.at[pl.ds(0, n)],
                            out_hbm.at[pl.ds(off, n)])

        cbase = jnp.where(wid < 26, 4 * wid, 104 + 3 * (wid - 26))

        def main_chunk(j, _):
            do_chunk((cbase + j) * CHUNK, CHUNK)
            return 0

        lax.fori_loop(0, 3, main_chunk, 0)

        @pl.when(wid < 26)
        def _():
            do_chunk((cbase + 3) * CHUNK, CHUNK)

        @pl.when(wid == NW - 1)
        def _():
            do_chunk(122 * CHUNK, TAIL)

    return k(heights_flat, findices)


def kernel(regions_oi, coordinates, local_region_ix, window, baseline_weight):
    table_t = baseline_weight.T
    ntile = (table_t.shape[1] // 128) * 128
    tail_cols = baseline_weight[ntile:, :]
    order = jnp.argsort(regions_oi)
    rsorted = jnp.take(regions_oi, order)
    pos = jnp.zeros_like(order).at[order].set(
        jnp.arange(order.shape[0], dtype=order.dtype))
    xsel = _sc_extract_columns(table_t, rsorted, tail_cols)
    heights_sorted = _tc_log_softmax(xsel, N_BINS)
    heights = _sc_gather_rows(heights_sorted, pos)
    heights_flat = heights.reshape(-1)
    c0 = coordinates[:, 0]
    findices = _tc_flat_indices(c0, local_region_ix, window)
    out = _sc_fragment_gather(heights_flat, findices)
    return out[:coordinates.shape[0]]
